# baseline (speedup 1.0000x reference)
import functools

import jax
import jax.numpy as jnp
from jax import lax
from jax.experimental import pallas as pl
from jax.experimental.pallas import tpu as pltpu
from jax.experimental.pallas import tpu_sc as plsc

_ATTRIBUTES_NUM = 8


@functools.lru_cache(maxsize=None)
def _make_sc_kernel(batch, seq_len, e_dims):
    info = plsc.get_sparse_core_info()
    nc, ns = info.num_cores, info.num_subcores
    nw = nc * ns
    b_per_w = batch // nw
    nrep = ns
    half = seq_len // 2
    chunk_rows = nrep * seq_len

    mesh = plsc.VectorSubcoreMesh(core_axis_name="c", subcore_axis_name="s")

    @functools.partial(
        pl.kernel,
        mesh=mesh,
        out_type=jax.ShapeDtypeStruct((batch * seq_len, e_dims), jnp.float32),
        scratch_types=[
            pltpu.VMEM((half,), jnp.int32),
            pltpu.VMEM((half,), jnp.int32),
            pltpu.VMEM((seq_len, e_dims), jnp.float32),
            pltpu.VMEM_SHARED((chunk_rows, e_dims), jnp.float32),
            pltpu.SemaphoreType.DMA,
            pltpu.SemaphoreType.DMA,
        ],
    )
    def sc_kernel(table_hbm, idx_lo_hbm, idx_hi_hbm, out_hbm,
                  idx_lo_v, idx_hi_v, rows_v, shared, gsem, ssem):
        cid = lax.axis_index("c")
        sid = lax.axis_index("s")
        wid = sid * nc + cid
        pltpu.sync_copy(idx_lo_hbm, idx_lo_v)
        pltpu.sync_copy(idx_hi_hbm, idx_hi_v)
        g0 = pltpu.async_copy(
            table_hbm.at[idx_lo_v], rows_v.at[pl.ds(0, half)], gsem)
        g1 = pltpu.async_copy(
            table_hbm.at[idx_hi_v], rows_v.at[pl.ds(half, half)], gsem)
        g0.wait()
        g1.wait()
        pltpu.sync_copy(rows_v, shared.at[pl.ds(sid * seq_len, seq_len)])
        plsc.subcore_barrier()
        base = wid * b_per_w * seq_len
        s0 = pltpu.async_copy(
            shared, out_hbm.at[pl.ds(base, chunk_rows)], ssem)
        s1 = pltpu.async_copy(
            shared, out_hbm.at[pl.ds(base + chunk_rows, chunk_rows)], ssem)
        s0.wait()
        s1.wait()

    return sc_kernel


def kernel(x, E_object_index):
    batch, seq_len = x.shape
    e_dims = E_object_index.shape[1]
    half = seq_len // 2
    idx = jnp.arange(seq_len, dtype=jnp.int32) // _ATTRIBUTES_NUM
    f = _make_sc_kernel(batch, seq_len, e_dims)
    flat = f(E_object_index, idx[:half], idx[half:])
    return flat.reshape(batch, seq_len, e_dims)


# speedup vs baseline: 1.6257x; 1.6257x over previous
import functools

import jax
import jax.numpy as jnp
from jax import lax
from jax.experimental import pallas as pl
from jax.experimental.pallas import tpu as pltpu
from jax.experimental.pallas import tpu_sc as plsc

_ATTRIBUTES_NUM = 8


@functools.lru_cache(maxsize=None)
def _make_sc_gather(seq_len, e_dims, table_rows):
    half = seq_len // 2
    mesh = plsc.VectorSubcoreMesh(core_axis_name="c", subcore_axis_name="s")

    @functools.partial(
        pl.kernel,
        mesh=mesh,
        out_type=jax.ShapeDtypeStruct((seq_len, e_dims), jnp.float32),
        scratch_types=[
            pltpu.VMEM((half,), jnp.int32),
            pltpu.VMEM((half,), jnp.int32),
            pltpu.VMEM((seq_len, e_dims), jnp.float32),
            pltpu.SemaphoreType.DMA,
            pltpu.SemaphoreType.DMA,
        ],
    )
    def sc_gather(table_hbm, idx_lo_hbm, idx_hi_hbm, seq_hbm,
                  idx_lo_v, idx_hi_v, rows_v, isem, gsem):
        wid = lax.axis_index("s") * 2 + lax.axis_index("c")

        @pl.when(wid == 0)
        def _():
            i0 = pltpu.async_copy(idx_lo_hbm, idx_lo_v, isem)
            i1 = pltpu.async_copy(idx_hi_hbm, idx_hi_v, isem)
            i0.wait()
            i1.wait()
            g0 = pltpu.async_copy(
                table_hbm.at[idx_lo_v], rows_v.at[pl.ds(0, half)], gsem)
            g1 = pltpu.async_copy(
                table_hbm.at[idx_hi_v], rows_v.at[pl.ds(half, half)], gsem)
            g0.wait()
            g1.wait()
            pltpu.sync_copy(rows_v, seq_hbm)

    return sc_gather


@functools.lru_cache(maxsize=None)
def _make_tc_broadcast(batch, seq_len, e_dims, k_rep):
    nchunks = batch // k_rep

    def body(seq_ref, out_ref, scratch_ref, sem):
        seq = seq_ref[:]
        for i in range(k_rep):
            scratch_ref[i] = seq
        copies = [
            pltpu.make_async_copy(
                scratch_ref,
                out_ref.at[pl.ds(c * k_rep, k_rep)],
                sem.at[c % 2],
            )
            for c in range(nchunks)
        ]
        for cp in copies:
            cp.start()
        for cp in copies:
            cp.wait()

    return pl.pallas_call(
        body,
        in_specs=[pl.BlockSpec(memory_space=pltpu.VMEM)],
        out_specs=pl.BlockSpec(memory_space=pltpu.MemorySpace.HBM),
        out_shape=jax.ShapeDtypeStruct((batch, seq_len, e_dims),
                                       jnp.float32),
        scratch_shapes=[
            pltpu.VMEM((k_rep, seq_len, e_dims), jnp.float32),
            pltpu.SemaphoreType.DMA((2,)),
        ],
    )


def kernel(x, E_object_index):
    batch, seq_len = x.shape
    table_rows, e_dims = E_object_index.shape
    half = seq_len // 2
    idx = jnp.arange(seq_len, dtype=jnp.int32) // _ATTRIBUTES_NUM
    gather = _make_sc_gather(seq_len, e_dims, table_rows)
    seq = gather(E_object_index, idx[:half], idx[half:])
    broadcast = _make_tc_broadcast(batch, seq_len, e_dims, k_rep=16)
    return broadcast(seq)
